# baseline (speedup 1.0000x reference)
import functools

import jax
import jax.numpy as jnp
from jax import lax
from jax.experimental import pallas as pl
from jax.experimental.pallas import tpu as pltpu


def _cdiv(a, b):
    return -(-a // b)


def _gather_kernel(idx_ref, wt_ref, out_ref, *, num_categories, groups, width):
    c = num_categories
    wt = wt_ref[...]
    idx = jnp.clip(idx_ref[...], 0, c - 1)
    iota_c = lax.broadcasted_iota(jnp.int32, (c, width), 0)
    for j in range(groups):
        row = idx[j:j + 1, :]
        onehot_t = (iota_c == row).astype(jnp.bfloat16)
        out_ref[:, pl.ds(width * j, width)] = jax.lax.dot_general(
            wt, onehot_t,
            dimension_numbers=(((1,), (0,)), ((), ())),
            preferred_element_type=jnp.float32,
        )


def kernel(category_indices, weight):
    C, D = weight.shape
    orig_shape = category_indices.shape

    idx = category_indices.reshape(-1).astype(jnp.int32)
    N = idx.shape[0]

    width = 256
    groups = 64
    tile = width * groups
    n_tiles = _cdiv(N, tile)
    Npad = n_tiles * tile
    if Npad != N:
        idx = jnp.pad(idx, (0, Npad - N))

    idx_2d = idx.reshape(n_tiles * groups, width)
    w_t = weight.T.astype(jnp.bfloat16)

    out_t = pl.pallas_call(
        functools.partial(_gather_kernel, num_categories=C,
                          groups=groups, width=width),
        out_shape=jax.ShapeDtypeStruct((D, N), jnp.float32),
        grid=(n_tiles,),
        in_specs=[
            pl.BlockSpec((groups, width), lambda i: (i, 0)),
            pl.BlockSpec((D, C), lambda i: (0, 0)),
        ],
        out_specs=pl.BlockSpec((D, tile), lambda i: (0, i)),
        compiler_params=pltpu.CompilerParams(
            dimension_semantics=("parallel",),
        ),
    )(idx_2d, w_t)

    out = jnp.transpose(out_t)
    return out.reshape(*orig_shape, D)


# speedup vs baseline: 1.3461x; 1.3461x over previous
import functools

import jax
import jax.numpy as jnp
from jax import lax
from jax.experimental import pallas as pl
from jax.experimental.pallas import tpu as pltpu


def _cdiv(a, b):
    return -(-a // b)


def _gather_kernel(idx_ref, wt_ref, out_ref, *, num_categories, chunks):
    c = num_categories
    wt = wt_ref[...]
    idx = jnp.clip(idx_ref[...], 0, c - 1)
    iota_c = lax.broadcasted_iota(jnp.int32, (c, 128), 0)
    for j in range(chunks):
        row = idx[j:j + 1, :]
        onehot_t = (iota_c == row).astype(jnp.bfloat16)
        out_ref[:, pl.ds(128 * j, 128)] = jax.lax.dot_general(
            wt, onehot_t,
            dimension_numbers=(((1,), (0,)), ((), ())),
            preferred_element_type=jnp.float32,
        )


def kernel(category_indices, weight):
    C, D = weight.shape
    orig_shape = category_indices.shape

    idx = category_indices.reshape(-1).astype(jnp.int32)
    N = idx.shape[0]

    chunks = 256
    tile = 128 * chunks
    n_tiles = _cdiv(N, tile)
    Npad = n_tiles * tile
    if Npad != N:
        idx = jnp.pad(idx, (0, Npad - N))

    idx_2d = idx.reshape(n_tiles * chunks, 128)
    w_t = weight.T.astype(jnp.bfloat16)

    out_t = pl.pallas_call(
        functools.partial(_gather_kernel, num_categories=C, chunks=chunks),
        out_shape=jax.ShapeDtypeStruct((D, N), jnp.float32),
        grid=(n_tiles,),
        in_specs=[
            pl.BlockSpec((chunks, 128), lambda i: (i, 0)),
            pl.BlockSpec((D, C), lambda i: (0, 0)),
        ],
        out_specs=pl.BlockSpec((D, tile), lambda i: (0, i)),
        compiler_params=pltpu.CompilerParams(
            dimension_semantics=("parallel",),
        ),
    )(idx_2d, w_t)

    out = jnp.transpose(out_t)
    return out.reshape(*orig_shape, D)


# speedup vs baseline: 1.3644x; 1.0136x over previous
import functools

import jax
import jax.numpy as jnp
from jax import lax
from jax.experimental import pallas as pl
from jax.experimental.pallas import tpu as pltpu


def _cdiv(a, b):
    return -(-a // b)


def _gather_kernel(idx_ref, wt_ref, out_ref, *, num_categories, chunks):
    c = num_categories
    wt = wt_ref[...]
    idx = jnp.clip(idx_ref[...], 0, c - 1)
    iota_c = lax.broadcasted_iota(jnp.int32, (c, 128), 0)
    for j in range(chunks):
        row = idx[j:j + 1, :]
        onehot_t = (iota_c == row).astype(jnp.bfloat16)
        out_ref[:, pl.ds(128 * j, 128)] = jax.lax.dot_general(
            wt, onehot_t,
            dimension_numbers=(((1,), (0,)), ((), ())),
            preferred_element_type=jnp.float32,
        )


def kernel(category_indices, weight):
    C, D = weight.shape
    orig_shape = category_indices.shape

    idx = category_indices.reshape(-1).astype(jnp.int32)
    N = idx.shape[0]

    chunks = 512
    tile = 128 * chunks
    n_tiles = _cdiv(N, tile)
    Npad = n_tiles * tile
    if Npad != N:
        idx = jnp.pad(idx, (0, Npad - N))

    idx_2d = idx.reshape(n_tiles * chunks, 128)
    w_t = weight.T.astype(jnp.bfloat16)

    out_t = pl.pallas_call(
        functools.partial(_gather_kernel, num_categories=C, chunks=chunks),
        out_shape=jax.ShapeDtypeStruct((D, N), jnp.float32),
        grid=(n_tiles,),
        in_specs=[
            pl.BlockSpec((chunks, 128), lambda i: (i, 0)),
            pl.BlockSpec((D, C), lambda i: (0, 0)),
        ],
        out_specs=pl.BlockSpec((D, tile), lambda i: (0, i)),
        compiler_params=pltpu.CompilerParams(
            dimension_semantics=("parallel",),
        ),
    )(idx_2d, w_t)

    out = jnp.transpose(out_t)
    return out.reshape(*orig_shape, D)
